# baseline (speedup 1.0000x reference)
import jax
import jax.numpy as jnp
from jax.experimental import pallas as pl
from jax.experimental.pallas import tpu as pltpu


_ROWS = 131072
_COLS = 128
_BLOCK_ROWS = 8192


def _copy_kernel(in_ref, out_ref):
    out_ref[...] = in_ref[...]


def kernel(stems):
    shape = stems.shape
    flat = stems.reshape(_ROWS, _COLS)
    out = pl.pallas_call(
        _copy_kernel,
        out_shape=jax.ShapeDtypeStruct(flat.shape, flat.dtype),
        grid=(_ROWS // _BLOCK_ROWS,),
        in_specs=[pl.BlockSpec((_BLOCK_ROWS, _COLS), lambda i: (i, 0))],
        out_specs=pl.BlockSpec((_BLOCK_ROWS, _COLS), lambda i: (i, 0)),
        compiler_params=pltpu.CompilerParams(
            dimension_semantics=("parallel",),
        ),
    )(flat)
    return out.reshape(shape)


# speedup vs baseline: 1.1950x; 1.1950x over previous
import functools

import jax
import jax.numpy as jnp
from jax import lax
from jax.experimental import pallas as pl
from jax.experimental.pallas import tpu as pltpu, tpu_sc as plsc


_N = 8 * 4 * 2 * 262144
_NC = 2
_NS = 16
_NW = _NC * _NS
_PER_W = _N // _NW
_C = 32768
_NCHUNK = _PER_W // _C

_mesh = plsc.VectorSubcoreMesh(core_axis_name="c", subcore_axis_name="s")


@functools.partial(
    pl.kernel,
    mesh=_mesh,
    out_type=jax.ShapeDtypeStruct((_N,), jnp.float32),
    scratch_types=[
        pltpu.VMEM((2, _C), jnp.float32),
        pltpu.SemaphoreType.DMA,
        pltpu.SemaphoreType.DMA,
        pltpu.SemaphoreType.DMA,
        pltpu.SemaphoreType.DMA,
    ],
)
def _sc_copy(in_hbm, out_hbm, buf, r0, r1, w0, w1):
    wid = lax.axis_index("s") * _NC + lax.axis_index("c")
    base = wid * _PER_W
    rsem = (r0, r1)
    wsem = (w0, w1)

    def read(i):
        s = i % 2
        return pltpu.make_async_copy(
            in_hbm.at[pl.ds(base + i * _C, _C)], buf.at[s], rsem[s])

    def write(i):
        s = i % 2
        return pltpu.make_async_copy(
            buf.at[s], out_hbm.at[pl.ds(base + i * _C, _C)], wsem[s])

    read(0).start()
    for i in range(_NCHUNK):
        read(i).wait()
        write(i).start()
        if i + 1 < _NCHUNK:
            if i >= 1:
                write(i - 1).wait()
            read(i + 1).start()
    if _NCHUNK >= 2:
        write(_NCHUNK - 2).wait()
    write(_NCHUNK - 1).wait()


def kernel(stems):
    shape = stems.shape
    out = _sc_copy(stems.reshape(_N))
    return out.reshape(shape)


# speedup vs baseline: 4.2732x; 3.5758x over previous
import functools

import jax
import jax.numpy as jnp
from jax import lax
from jax.experimental import pallas as pl
from jax.experimental.pallas import tpu as pltpu, tpu_sc as plsc


_SHAPE = (8, 4, 2, 262144)
_L = _SHAPE[-1]
_NROWS = 8 * 4 * 2
_NC = 2
_NS = 16
_NW = _NC * _NS
_ROWS_PER_W = _NROWS // _NW
_C = 32768
_CPR = _L // _C

_mesh = plsc.VectorSubcoreMesh(core_axis_name="c", subcore_axis_name="s")


@functools.partial(
    pl.kernel,
    mesh=_mesh,
    out_type=jax.ShapeDtypeStruct(_SHAPE, jnp.float32),
    scratch_types=[
        pltpu.VMEM((2, _C), jnp.float32),
        pltpu.SemaphoreType.DMA,
        pltpu.SemaphoreType.DMA,
        pltpu.SemaphoreType.DMA,
        pltpu.SemaphoreType.DMA,
    ],
)
def _sc_copy(in_hbm, out_hbm, buf, r0, r1, w0, w1):
    wid = lax.axis_index("s") * _NC + lax.axis_index("c")
    rsem = (r0, r1)
    wsem = (w0, w1)

    def row_idx(j):
        row = wid * _ROWS_PER_W + j
        return row >> 3, (row >> 1) & 3, row & 1

    def read(i):
        s = i % 2
        b, st, ch = row_idx(i // _CPR)
        off = (i % _CPR) * _C
        return pltpu.make_async_copy(
            in_hbm.at[b, st, ch, pl.ds(off, _C)], buf.at[s], rsem[s])

    def write(i):
        s = i % 2
        b, st, ch = row_idx(i // _CPR)
        off = (i % _CPR) * _C
        return pltpu.make_async_copy(
            buf.at[s], out_hbm.at[b, st, ch, pl.ds(off, _C)], wsem[s])

    n = _ROWS_PER_W * _CPR
    read(0).start()
    for i in range(n):
        read(i).wait()
        write(i).start()
        if i + 1 < n:
            if i >= 1:
                write(i - 1).wait()
            read(i + 1).start()
    write(n - 2).wait()
    write(n - 1).wait()


def kernel(stems):
    return _sc_copy(stems)


# speedup vs baseline: 4.8136x; 1.1265x over previous
import jax
import jax.numpy as jnp
from jax.experimental import pallas as pl
from jax.experimental.pallas import tpu as pltpu


_SHAPE = (8, 4, 2, 262144)
_L = _SHAPE[-1]
_NROWS = 64
_S = 6


def _copy_kernel(in_ref, out_ref, *scratch):
    bufs = scratch[:_S]
    rsems = scratch[_S:2 * _S]
    wsems = scratch[2 * _S:3 * _S]

    def row_idx(r):
        return r >> 3, (r >> 1) & 3, r & 1

    def read(i):
        s = i % _S
        b, st, ch = row_idx(i)
        return pltpu.make_async_copy(in_ref.at[b, st, ch], bufs[s], rsems[s])

    def write(i):
        s = i % _S
        b, st, ch = row_idx(i)
        return pltpu.make_async_copy(bufs[s], out_ref.at[b, st, ch], wsems[s])

    for k in range(_S):
        read(k).start()
    for i in range(_NROWS):
        if i >= 1 and i - 1 + _S < _NROWS:
            write(i - 1).wait()
            read(i - 1 + _S).start()
        read(i).wait()
        write(i).start()
    for i in range(_NROWS - _S, _NROWS):
        write(i).wait()


def tc_copy(stems):
    return pl.pallas_call(
        _copy_kernel,
        out_shape=jax.ShapeDtypeStruct(_SHAPE, jnp.float32),
        in_specs=[pl.BlockSpec(memory_space=pltpu.MemorySpace.HBM)],
        out_specs=pl.BlockSpec(memory_space=pltpu.MemorySpace.HBM),
        scratch_shapes=(
            [pltpu.VMEM((_L,), jnp.float32)] * _S
            + [pltpu.SemaphoreType.DMA] * (2 * _S)
        ),
    )(stems)


def kernel(stems):
    return tc_copy(stems)


# speedup vs baseline: 5.5461x; 1.1522x over previous
import jax
import jax.numpy as jnp
from jax.experimental import pallas as pl
from jax.experimental.pallas import tpu as pltpu


_SHAPE = (8, 4, 2, 262144)
_L = _SHAPE[-1]
_NROWS = 64
_S = 16


def _copy_kernel(in_ref, out_ref, *scratch):
    bufs = scratch[:_S]
    rsems = scratch[_S:2 * _S]
    wsems = scratch[2 * _S:3 * _S]

    def row_idx(r):
        return r >> 3, (r >> 1) & 3, r & 1

    def read(i):
        s = i % _S
        b, st, ch = row_idx(i)
        return pltpu.make_async_copy(in_ref.at[b, st, ch], bufs[s], rsems[s])

    def write(i):
        s = i % _S
        b, st, ch = row_idx(i)
        return pltpu.make_async_copy(bufs[s], out_ref.at[b, st, ch], wsems[s])

    for k in range(_S):
        read(k).start()
    for i in range(_NROWS):
        if i >= 1 and i - 1 + _S < _NROWS:
            write(i - 1).wait()
            read(i - 1 + _S).start()
        read(i).wait()
        write(i).start()
    for i in range(_NROWS - _S, _NROWS):
        write(i).wait()


def tc_copy(stems):
    return pl.pallas_call(
        _copy_kernel,
        out_shape=jax.ShapeDtypeStruct(_SHAPE, jnp.float32),
        in_specs=[pl.BlockSpec(memory_space=pltpu.MemorySpace.HBM)],
        out_specs=pl.BlockSpec(memory_space=pltpu.MemorySpace.HBM),
        scratch_shapes=(
            [pltpu.VMEM((_L,), jnp.float32)] * _S
            + [pltpu.SemaphoreType.DMA] * (2 * _S)
        ),
    )(stems)


def kernel(stems):
    return tc_copy(stems)


# speedup vs baseline: 6.7727x; 1.2212x over previous
import jax
import jax.numpy as jnp
from jax.experimental import pallas as pl
from jax.experimental.pallas import tpu as pltpu


_SHAPE = (8, 4, 2, 262144)
_L = _SHAPE[-1]
_NROWS = 64
_S = 32


def _copy_kernel(in_ref, out_ref, *scratch):
    bufs = scratch[:_S]
    rsems = scratch[_S:2 * _S]
    wsems = scratch[2 * _S:3 * _S]

    def row_idx(r):
        return r >> 3, (r >> 1) & 3, r & 1

    def read(i):
        s = i % _S
        b, st, ch = row_idx(i)
        return pltpu.make_async_copy(in_ref.at[b, st, ch], bufs[s], rsems[s])

    def write(i):
        s = i % _S
        b, st, ch = row_idx(i)
        return pltpu.make_async_copy(bufs[s], out_ref.at[b, st, ch], wsems[s])

    for k in range(_S):
        read(k).start()
    for i in range(_NROWS):
        if i >= 1 and i - 1 + _S < _NROWS:
            write(i - 1).wait()
            read(i - 1 + _S).start()
        read(i).wait()
        write(i).start()
    for i in range(_NROWS - _S, _NROWS):
        write(i).wait()


def tc_copy(stems):
    return pl.pallas_call(
        _copy_kernel,
        out_shape=jax.ShapeDtypeStruct(_SHAPE, jnp.float32),
        in_specs=[pl.BlockSpec(memory_space=pltpu.MemorySpace.HBM)],
        out_specs=pl.BlockSpec(memory_space=pltpu.MemorySpace.HBM),
        scratch_shapes=(
            [pltpu.VMEM((_L,), jnp.float32)] * _S
            + [pltpu.SemaphoreType.DMA] * (2 * _S)
        ),
    )(stems)


def kernel(stems):
    return tc_copy(stems)


# speedup vs baseline: 7.9711x; 1.1769x over previous
import jax
import jax.numpy as jnp
from jax.experimental import pallas as pl
from jax.experimental.pallas import tpu as pltpu


_SHAPE = (8, 4, 2, 262144)
_L = _SHAPE[-1]
_NROWS = 64
_S = 48


def _copy_kernel(in_ref, out_ref, *scratch):
    bufs = scratch[:_S]
    rsems = scratch[_S:2 * _S]
    wsems = scratch[2 * _S:3 * _S]

    def row_idx(r):
        return r >> 3, (r >> 1) & 3, r & 1

    def read(i):
        s = i % _S
        b, st, ch = row_idx(i)
        return pltpu.make_async_copy(in_ref.at[b, st, ch], bufs[s], rsems[s])

    def write(i):
        s = i % _S
        b, st, ch = row_idx(i)
        return pltpu.make_async_copy(bufs[s], out_ref.at[b, st, ch], wsems[s])

    for k in range(_S):
        read(k).start()
    for i in range(_NROWS):
        if i >= 1 and i - 1 + _S < _NROWS:
            write(i - 1).wait()
            read(i - 1 + _S).start()
        read(i).wait()
        write(i).start()
    for i in range(_NROWS - _S, _NROWS):
        write(i).wait()


def tc_copy(stems):
    return pl.pallas_call(
        _copy_kernel,
        out_shape=jax.ShapeDtypeStruct(_SHAPE, jnp.float32),
        in_specs=[pl.BlockSpec(memory_space=pltpu.MemorySpace.HBM)],
        out_specs=pl.BlockSpec(memory_space=pltpu.MemorySpace.HBM),
        scratch_shapes=(
            [pltpu.VMEM((_L,), jnp.float32)] * _S
            + [pltpu.SemaphoreType.DMA] * (2 * _S)
        ),
    )(stems)


def kernel(stems):
    return tc_copy(stems)
